# baseline (speedup 1.0000x reference)
import functools

import jax
import jax.numpy as jnp
from jax.experimental import pallas as pl
from jax.experimental.pallas import tpu as pltpu

_K = 1024
_C = 32
_CCOST = 0.025


def _vq_block(x_ref, emb_ref, out_ref, idx_ref, acc_ref,
              emb2_s, esq_s, ehi_s, elo_s, *, bm, nb, nj):
    b = pl.program_id(0)
    j = pl.program_id(1)

    @pl.when((b == 0) & (j == 0))
    def _init():
        e = emb_ref[...]
        emb2_s[...] = e + e
        esq_s[...] = jnp.sum(e * e, axis=1, keepdims=True)
        eT = e.T
        hi = eT.astype(jnp.bfloat16)
        ehi_s[...] = hi
        elo_s[...] = (eT - hi.astype(jnp.float32)).astype(jnp.bfloat16)
        acc_ref[...] = jnp.zeros_like(acc_ref)

    xb = x_ref[0]
    xsq = jnp.sum(xb * xb, axis=0, keepdims=True)
    mmT = jax.lax.dot_general(
        emb2_s[...], xb, (((1,), (0,)), ((), ())),
        preferred_element_type=jnp.float32)
    d = (xsq + esq_s[...]) - mmT
    dmin = jnp.min(d, axis=0, keepdims=True)
    kio = jax.lax.broadcasted_iota(jnp.int32, (_K, bm), 0)
    isel = jnp.where(d == dmin, kio, _K)
    idx = jnp.min(isel, axis=0)
    onehot = (isel == idx[None, :]).astype(jnp.bfloat16)
    qT = (jax.lax.dot_general(
              ehi_s[...], onehot, (((1,), (0,)), ((), ())),
              preferred_element_type=jnp.float32)
          + jax.lax.dot_general(
              elo_s[...], onehot, (((1,), (0,)), ((), ())),
              preferred_element_type=jnp.float32))
    out_ref[0] = qT
    idx_ref[0, 0] = idx
    acc_ref[...] += jnp.sum(dmin, axis=1, keepdims=True)

    @pl.when((b == nb - 1) & (j == nj - 1))
    def _fin():
        m = acc_ref[...] * (1.0 / (nb * nj * bm * _C))
        acc_ref[...] = m + _CCOST * m


def kernel(x, embeddings):
    B, C, D, H, W = x.shape
    M = D * H * W
    x3 = x.reshape(B, C, M)
    bm = 2048
    nj = M // bm
    out3, idx3, acc = pl.pallas_call(
        functools.partial(_vq_block, bm=bm, nb=B, nj=nj),
        grid=(B, nj),
        in_specs=[
            pl.BlockSpec((1, C, bm), lambda b, j: (b, 0, j)),
            pl.BlockSpec((_K, _C), lambda b, j: (0, 0)),
        ],
        out_specs=[
            pl.BlockSpec((1, C, bm), lambda b, j: (b, 0, j)),
            pl.BlockSpec((1, 1, bm), lambda b, j: (b, 0, j)),
            pl.BlockSpec((1, 1), lambda b, j: (0, 0)),
        ],
        out_shape=[
            jax.ShapeDtypeStruct((B, C, M), jnp.float32),
            jax.ShapeDtypeStruct((B, 1, M), jnp.int32),
            jax.ShapeDtypeStruct((1, 1), jnp.float32),
        ],
        scratch_shapes=[
            pltpu.VMEM((_K, _C), jnp.float32),
            pltpu.VMEM((_K, 1), jnp.float32),
            pltpu.VMEM((_C, _K), jnp.bfloat16),
            pltpu.VMEM((_C, _K), jnp.bfloat16),
        ],
    )(x3, embeddings)
    out = out3.reshape(B, C, D, H, W)
    indices = idx3.reshape(B, D, H, W)
    loss = acc[0, 0]
    return (out, loss, indices)


# speedup vs baseline: 1.1583x; 1.1583x over previous
import functools

import jax
import jax.numpy as jnp
from jax.experimental import pallas as pl
from jax.experimental.pallas import tpu as pltpu

_K = 1024
_C = 32
_CCOST = 0.025


def _vq_block(x_ref, emb_ref, out_ref, idx_ref, acc_ref,
              emb2_s, esq_s, embT_s, *, bm, nb, nj):
    b = pl.program_id(0)
    j = pl.program_id(1)

    @pl.when((b == 0) & (j == 0))
    def _init():
        e = emb_ref[...]
        emb2_s[...] = e + e
        esq_s[...] = jnp.sum(e * e, axis=1, keepdims=True)
        embT_s[...] = e.T
        acc_ref[...] = jnp.zeros_like(acc_ref)

    xb = x_ref[0]
    xsq = jnp.sum(xb * xb, axis=0, keepdims=True)
    mmT = jax.lax.dot_general(
        emb2_s[...], xb, (((1,), (0,)), ((), ())),
        preferred_element_type=jnp.float32)
    d = (xsq + esq_s[...]) - mmT
    dmin = jnp.min(d, axis=0, keepdims=True)
    kio = jax.lax.broadcasted_iota(jnp.int32, (_K, bm), 0)
    isel = jnp.where(d == dmin, kio, _K)
    idx = jnp.min(isel, axis=0)
    onehot = (isel == idx[None, :]).astype(jnp.float32)
    qT = jax.lax.dot_general(
        embT_s[...], onehot, (((1,), (0,)), ((), ())),
        preferred_element_type=jnp.float32)
    out_ref[0] = qT
    idx_ref[0, 0] = idx
    acc_ref[...] += jnp.sum(dmin, axis=1, keepdims=True)

    @pl.when((b == nb - 1) & (j == nj - 1))
    def _fin():
        m = acc_ref[...] * (1.0 / (nb * nj * bm * _C))
        acc_ref[...] = m + _CCOST * m


def kernel(x, embeddings):
    B, C, D, H, W = x.shape
    M = D * H * W
    x3 = x.reshape(B, C, M)
    bm = 2048
    nj = M // bm
    out3, idx3, acc = pl.pallas_call(
        functools.partial(_vq_block, bm=bm, nb=B, nj=nj),
        grid=(B, nj),
        in_specs=[
            pl.BlockSpec((1, C, bm), lambda b, j: (b, 0, j)),
            pl.BlockSpec((_K, _C), lambda b, j: (0, 0)),
        ],
        out_specs=[
            pl.BlockSpec((1, C, bm), lambda b, j: (b, 0, j)),
            pl.BlockSpec((1, 1, bm), lambda b, j: (b, 0, j)),
            pl.BlockSpec((1, 1), lambda b, j: (0, 0)),
        ],
        out_shape=[
            jax.ShapeDtypeStruct((B, C, M), jnp.float32),
            jax.ShapeDtypeStruct((B, 1, M), jnp.int32),
            jax.ShapeDtypeStruct((1, 1), jnp.float32),
        ],
        scratch_shapes=[
            pltpu.VMEM((_K, _C), jnp.float32),
            pltpu.VMEM((_K, 1), jnp.float32),
            pltpu.VMEM((_C, _K), jnp.float32),
        ],
    )(x3, embeddings)
    out = out3.reshape(B, C, D, H, W)
    indices = idx3.reshape(B, D, H, W)
    loss = acc[0, 0]
    return (out, loss, indices)


# speedup vs baseline: 1.4428x; 1.2456x over previous
import functools

import jax
import jax.numpy as jnp
from jax.experimental import pallas as pl
from jax.experimental.pallas import tpu as pltpu

_K = 1024
_C = 32
_CCOST = 0.025


def _vq_block(x_ref, emb_ref, out_ref, idx_ref, acc_ref,
              emb2_s, esq_s, ek_s, *, bm, nb, nj):
    b = pl.program_id(0)
    j = pl.program_id(1)

    @pl.when((b == 0) & (j == 0))
    def _init():
        e = emb_ref[...]
        emb2_s[...] = e + e
        esq_s[...] = jnp.sum(e * e, axis=1, keepdims=True)
        ek_s[...] = e.reshape(_K // 8, 8, _C).transpose(1, 2, 0).reshape(8 * _C, _K // 8)
        acc_ref[...] = jnp.zeros_like(acc_ref)

    xb = x_ref[0]
    xsq = jnp.sum(xb * xb, axis=0, keepdims=True)
    mmT = jax.lax.dot_general(
        emb2_s[...], xb, (((1,), (0,)), ((), ())),
        preferred_element_type=jnp.float32)
    d = (xsq + esq_s[...]) - mmT
    dc = d.reshape(_K // 8, 8, bm)
    ri = None
    step = 1
    while dc.shape[0] > 1:
        half = dc.shape[0] // 2
        d4 = dc.reshape(half, 2, 8, bm)
        da, db = d4[:, 0], d4[:, 1]
        take = db < da
        if ri is None:
            ri = jnp.where(take, 1, 0)
        else:
            r4 = ri.reshape(half, 2, 8, bm)
            ri = jnp.where(take, r4[:, 1] + step, r4[:, 0])
        dc = jnp.where(take, db, da)
        step *= 2
    dmin8 = dc[0]
    k8 = ri[0] * 8 + jax.lax.broadcasted_iota(jnp.int32, (8, bm), 0)
    dd, kk = dmin8, k8
    h = 4
    while h >= 1:
        tlt = (dd[h:] < dd[:h]) | ((dd[h:] == dd[:h]) & (kk[h:] < kk[:h]))
        dd = jnp.where(tlt, dd[h:], dd[:h])
        kk = jnp.where(tlt, kk[h:], kk[:h])
        h //= 2
    dmin = dd
    idx = kk[0]
    rowf = kk // 8
    subf = kk - rowf * 8
    rio = jax.lax.broadcasted_iota(jnp.int32, (_K // 8, bm), 0)
    rowhot = (rio == rowf).astype(jnp.float32)
    gall = jax.lax.dot_general(
        ek_s[...], rowhot, (((1,), (0,)), ((), ())),
        preferred_element_type=jnp.float32)
    g3 = gall.reshape(8, _C, bm)
    qT = None
    for sb in range(8):
        term = g3[sb] * (subf == sb).astype(jnp.float32)
        qT = term if qT is None else qT + term
    out_ref[0] = qT
    idx_ref[0, 0] = idx
    acc_ref[...] += jnp.sum(dmin, axis=1, keepdims=True)

    @pl.when((b == nb - 1) & (j == nj - 1))
    def _fin():
        m = acc_ref[...] * (1.0 / (nb * nj * bm * _C))
        acc_ref[...] = m + _CCOST * m


def kernel(x, embeddings):
    B, C, D, H, W = x.shape
    M = D * H * W
    x3 = x.reshape(B, C, M)
    bm = 2048
    nj = M // bm
    out3, idx3, acc = pl.pallas_call(
        functools.partial(_vq_block, bm=bm, nb=B, nj=nj),
        grid=(B, nj),
        in_specs=[
            pl.BlockSpec((1, C, bm), lambda b, j: (b, 0, j)),
            pl.BlockSpec((_K, _C), lambda b, j: (0, 0)),
        ],
        out_specs=[
            pl.BlockSpec((1, C, bm), lambda b, j: (b, 0, j)),
            pl.BlockSpec((1, 1, bm), lambda b, j: (b, 0, j)),
            pl.BlockSpec((1, 1), lambda b, j: (0, 0)),
        ],
        out_shape=[
            jax.ShapeDtypeStruct((B, C, M), jnp.float32),
            jax.ShapeDtypeStruct((B, 1, M), jnp.int32),
            jax.ShapeDtypeStruct((1, 1), jnp.float32),
        ],
        scratch_shapes=[
            pltpu.VMEM((_K, _C), jnp.float32),
            pltpu.VMEM((_K, 1), jnp.float32),
            pltpu.VMEM((8 * _C, _K // 8), jnp.float32),
        ],
    )(x3, embeddings)
    out = out3.reshape(B, C, D, H, W)
    indices = idx3.reshape(B, D, H, W)
    loss = acc[0, 0]
    return (out, loss, indices)
